# baseline (speedup 1.0000x reference)
import functools
import jax
import jax.numpy as jnp
from jax import lax
from jax.experimental import pallas as pl
from jax.experimental.pallas import tpu as pltpu
from jax.experimental.pallas import tpu_sc as plsc

N = 10000
E = 320000
NS = 16
NP = 10240
RP = NP // NS
RP4 = RP * 4
EPW = E // NS
EC = 2000
NCHUNK = EPW // EC

_mesh = plsc.VectorSubcoreMesh(core_axis_name="c", subcore_axis_name="s",
                               num_cores=1)


def _mm_body(x_ref, w_ref, o_ref):
    o_ref[...] = jnp.dot(x_ref[...], w_ref[...],
                         preferred_element_type=jnp.float32)


def _rsqrt16(x):
    i = plsc.bitcast(x, jnp.int32)
    y = plsc.bitcast(jnp.int32(0x5F3759DF) - (i >> 1), jnp.float32)
    for _ in range(4):
        y = y * (1.5 - 0.5 * x * y * y)
    return y


def _tanh16(x):
    e = jnp.exp(2.0 * x)
    return 1.0 - 2.0 / (e + 1.0)


@functools.partial(
    pl.kernel,
    out_type=[
        jax.ShapeDtypeStruct((NP * 4,), jnp.float32),
        jax.ShapeDtypeStruct((NP * 2,), jnp.float32),
    ],
    mesh=_mesh,
    compiler_params=pltpu.CompilerParams(needs_layout_passes=False),
    scratch_types=[
        pltpu.VMEM_SHARED((NP * 4,), jnp.float32),
        pltpu.VMEM_SHARED((NP * 4,), jnp.float32),
        pltpu.VMEM_SHARED((NP,), jnp.float32),
        [pltpu.VMEM((EC * 4,), jnp.int32)] * 2,
        [pltpu.VMEM((EC * 4,), jnp.int32)] * 2,
        [pltpu.VMEM((EC * 4,), jnp.float32)] * 2,
        [pltpu.VMEM((EC * 2,), jnp.int32)] * 2,
        [pltpu.VMEM((EC * 2,), jnp.int32)] * 2,
        [pltpu.VMEM((EC * 2,), jnp.float32)] * 2,
        pltpu.VMEM((EC,), jnp.int32),
        pltpu.VMEM((EC,), jnp.float32),
        pltpu.VMEM((RP4,), jnp.float32),
        pltpu.VMEM((RP4,), jnp.float32),
        pltpu.VMEM((RP4,), jnp.float32),
        pltpu.VMEM((RP4,), jnp.float32),
        pltpu.VMEM((RP,), jnp.float32),
        pltpu.VMEM((64,), jnp.float32),
        [pltpu.SemaphoreType.DMA] * 6,
        pltpu.SemaphoreType.DMA,
    ],
)
def _gcn_sc(dst_hbm, src4_hbm, dst4_hbm, src2_hbm, dst2_hbm, h1_hbm, par_hbm,
            out_hbm, hout_hbm,
            g_sp, acc_sp, deg_sp, gi_v, si_v, msg_v, gi2_v, si2_v, msg2_v,
            di_v, one_v, z_v, g_v, a_v, h_v, d_v, p_v, sems, sem):
    wid = lax.axis_index("s")
    nsl = pl.ds(wid * RP4, RP4)
    lanes = lax.iota(jnp.int32, 16)

    def _fill(i, _):
        one_v[pl.ds(i * 16, 16)] = jnp.full((16,), 1.0, jnp.float32)
        return _
    lax.fori_loop(0, EC // 16, _fill, None)

    def _zfill(i, _):
        z_v[pl.ds(i * 16, 16)] = jnp.zeros((16,), jnp.float32)
        return _
    lax.fori_loop(0, RP4 // 16, _zfill, None)

    pltpu.sync_copy(par_hbm, p_v)
    pltpu.sync_copy(z_v.at[pl.ds(0, RP)], deg_sp.at[pl.ds(wid * RP, RP)])
    pltpu.sync_copy(z_v, acc_sp.at[nsl])
    plsc.subcore_barrier()

    for c in range(NCHUNK):
        pltpu.sync_copy(dst_hbm.at[pl.ds(wid * EPW + c * EC, EC)], di_v)
        pltpu.sync_copy(one_v, deg_sp.at[di_v], add=True)
    plsc.subcore_barrier()

    pltpu.sync_copy(deg_sp.at[pl.ds(wid * RP, RP)], d_v)

    def _dinv(i, _):
        s = pl.ds(i * 16, 16)
        d_v[s] = _rsqrt16(d_v[s] + 1.0)
        return _
    lax.fori_loop(0, RP // 16, _dinv, None)

    pltpu.sync_copy(h1_hbm.at[nsl], g_v)

    def _scale(i, _):
        s = pl.ds(i * 16, 16)
        dv = plsc.load_gather(d_v, [(lanes + i * 16) >> 2])
        g_v[s] = g_v[s] * dv
        return _
    lax.fori_loop(0, RP4 // 16, _scale, None)

    pltpu.sync_copy(g_v, g_sp.at[nsl])
    plsc.subcore_barrier()

    def _edge_pass(srci_hbm, dsti_hbm, gi, si, msg, w, sg, sd, mult):
        cl = EC * mult

        def sl(c):
            return pl.ds((wid * EPW + c * EC) * mult, cl)

        pltpu.async_copy(srci_hbm.at[sl(0)], gi[0], w[0]).wait()
        gd = pltpu.async_copy(g_sp.at[gi[0]], msg[0], sg[0])
        dd = pltpu.async_copy(dsti_hbm.at[sl(0)], si[0], sd[0])
        for c in range(NCHUNK):
            b = c % 2
            nb = 1 - b
            if c + 1 < NCHUNK:
                wn = pltpu.async_copy(srci_hbm.at[sl(c + 1)], gi[nb], w[nb])
                dn = pltpu.async_copy(dsti_hbm.at[sl(c + 1)], si[nb], sd[nb])
            gd.wait()
            if c + 1 < NCHUNK:
                wn.wait()
                gd = pltpu.async_copy(g_sp.at[gi[nb]], msg[nb], sg[nb])
            dd.wait()
            pltpu.sync_copy(msg[b], acc_sp.at[si[b]], add=True)
            if c + 1 < NCHUNK:
                dd = dn

    sw = sems[0:2]
    sd_ = sems[2:4]
    sg_ = sems[4:6]

    for layer in range(3):
        if layer < 2:
            _edge_pass(src4_hbm, dst4_hbm, gi_v, si_v, msg_v,
                       sw, sg_, sd_, 4)
        else:
            _edge_pass(src2_hbm, dst2_hbm, gi2_v, si2_v, msg2_v,
                       sw, sg_, sd_, 2)
        plsc.subcore_barrier()

        pltpu.sync_copy(acc_sp.at[nsl], a_v)
        boff = 48 + 4 * layer

        def _node(i, _):
            s = pl.ds(i * 16, 16)
            flat = lanes + i * 16
            dv = plsc.load_gather(d_v, [flat >> 2])
            bv = plsc.load_gather(p_v, [boff + (flat & 3)])
            h_v[s] = _tanh16(dv * (a_v[s] + g_v[s]) + bv)
            return _
        lax.fori_loop(0, RP4 // 16, _node, None)

        if layer < 2:
            woff = 16 * layer

            def _mm(i, _):
                s = pl.ds(i * 16, 16)
                flat = lanes + i * 16
                nd4 = (flat >> 2) << 2
                cc = flat & 3
                acc = jnp.zeros((16,), jnp.float32)
                for k in range(4):
                    hk = plsc.load_gather(h_v, [nd4 + k])
                    wk = plsc.load_gather(p_v, [woff + 4 * k + cc])
                    acc = acc + hk * wk
                dv = plsc.load_gather(d_v, [flat >> 2])
                g_v[s] = acc * dv
                return _
            lax.fori_loop(0, RP4 // 16, _mm, None)

            pltpu.sync_copy(g_v, g_sp.at[nsl])
            pltpu.sync_copy(z_v, acc_sp.at[nsl])
        else:
            def _cls(i, _):
                s = pl.ds(i * 16, 16)
                flat = lanes + i * 16
                nd4 = (flat >> 2) << 2
                cc = flat & 3
                acc = plsc.load_gather(p_v, [60 + cc])
                for k in range(4):
                    hk = plsc.load_gather(h_v, [nd4 + k])
                    wk = plsc.load_gather(p_v, [32 + 4 * k + cc])
                    acc = acc + hk * wk
                a_v[s] = acc
                return _
            lax.fori_loop(0, RP4 // 16, _cls, None)

            pltpu.sync_copy(a_v, out_hbm.at[nsl])

            def _hout(i, _):
                s = pl.ds(i * 16, 16)
                f2 = lanes + i * 16
                g_v[s] = plsc.load_gather(h_v, [((f2 >> 1) << 2) + (f2 & 1)])
                return _
            lax.fori_loop(0, RP * 2 // 16, _hout, None)

            pltpu.sync_copy(g_v.at[pl.ds(0, RP * 2)],
                            hout_hbm.at[pl.ds(wid * RP * 2, RP * 2)])
        plsc.subcore_barrier()


def kernel(x, edge_index, W1, b1, W2, b2, W3, b3, Wc, bc):
    src = edge_index[0]
    dst = edge_index[1]

    h1 = pl.pallas_call(
        _mm_body,
        out_shape=jax.ShapeDtypeStruct((N, 4), jnp.float32),
    )(x, W1)

    h1f = jnp.pad(h1, ((0, NP - N), (0, 0))).ravel()
    four = jnp.arange(4, dtype=jnp.int32)
    two = jnp.arange(2, dtype=jnp.int32)
    src4 = (4 * src[:, None] + four).ravel()
    dst4 = (4 * dst[:, None] + four).ravel()
    src2 = (4 * src[:, None] + two).ravel()
    dst2 = (4 * dst[:, None] + two).ravel()
    w3p = jnp.pad(W3, ((0, 0), (0, 2)))
    wcp = jnp.pad(Wc, ((0, 2), (0, 0)))
    b3p = jnp.pad(b3, (0, 2))
    par = jnp.concatenate([W2.ravel(), w3p.ravel(), wcp.ravel(),
                           b1, b2, b3p, bc]).astype(jnp.float32)

    out_f, h_f = _gcn_sc(dst, src4, dst4, src2, dst2, h1f, par)
    out = out_f.reshape(NP, 4)[:N]
    h = h_f.reshape(NP, 2)[:N]
    return (out, h)


# speedup vs baseline: 1.0048x; 1.0048x over previous
import functools
import jax
import jax.numpy as jnp
from jax import lax
from jax.experimental import pallas as pl
from jax.experimental.pallas import tpu as pltpu
from jax.experimental.pallas import tpu_sc as plsc

N = 10000
E = 320000
NS = 16
NP = 10240
RP = NP // NS
RP4 = RP * 4
EPW = E // NS
EC = 4000
NCHUNK = EPW // EC

_mesh = plsc.VectorSubcoreMesh(core_axis_name="c", subcore_axis_name="s",
                               num_cores=1)


def _mm_body(x_ref, w_ref, o_ref):
    o_ref[...] = jnp.dot(x_ref[...], w_ref[...],
                         preferred_element_type=jnp.float32)


def _rsqrt16(x):
    i = plsc.bitcast(x, jnp.int32)
    y = plsc.bitcast(jnp.int32(0x5F3759DF) - (i >> 1), jnp.float32)
    for _ in range(4):
        y = y * (1.5 - 0.5 * x * y * y)
    return y


def _tanh16(x):
    e = jnp.exp(2.0 * x)
    return 1.0 - 2.0 / (e + 1.0)


@functools.partial(
    pl.kernel,
    out_type=[
        jax.ShapeDtypeStruct((NP * 4,), jnp.float32),
        jax.ShapeDtypeStruct((NP * 2,), jnp.float32),
    ],
    mesh=_mesh,
    compiler_params=pltpu.CompilerParams(needs_layout_passes=False),
    scratch_types=[
        pltpu.VMEM_SHARED((NP * 4,), jnp.float32),
        pltpu.VMEM_SHARED((NP * 4,), jnp.float32),
        pltpu.VMEM_SHARED((NP,), jnp.float32),
        pltpu.VMEM((EC * 4,), jnp.int32),
        pltpu.VMEM((EC * 4,), jnp.int32),
        pltpu.VMEM((EC * 4,), jnp.float32),
        pltpu.VMEM((EC * 2,), jnp.int32),
        pltpu.VMEM((EC * 2,), jnp.int32),
        pltpu.VMEM((EC * 2,), jnp.float32),
        pltpu.VMEM((EC,), jnp.int32),
        pltpu.VMEM((EC,), jnp.float32),
        pltpu.VMEM((RP4,), jnp.float32),
        pltpu.VMEM((RP4,), jnp.float32),
        pltpu.VMEM((RP4,), jnp.float32),
        pltpu.VMEM((RP4,), jnp.float32),
        pltpu.VMEM((RP,), jnp.float32),
        pltpu.VMEM((64,), jnp.float32),
        pltpu.SemaphoreType.DMA,
    ],
)
def _gcn_sc(dst_hbm, src4_hbm, dst4_hbm, src2_hbm, dst2_hbm, h1_hbm, par_hbm,
            out_hbm, hout_hbm,
            g_sp, acc_sp, deg_sp, gi_v, si_v, msg_v, gi2_v, si2_v, msg2_v,
            di_v, one_v, z_v, g_v, a_v, h_v, d_v, p_v, sem):
    wid = lax.axis_index("s")
    nsl = pl.ds(wid * RP4, RP4)
    lanes = lax.iota(jnp.int32, 16)

    def _fill(i, _):
        one_v[pl.ds(i * 16, 16)] = jnp.full((16,), 1.0, jnp.float32)
        return _
    lax.fori_loop(0, EC // 16, _fill, None)

    def _zfill(i, _):
        z_v[pl.ds(i * 16, 16)] = jnp.zeros((16,), jnp.float32)
        return _
    lax.fori_loop(0, RP4 // 16, _zfill, None)

    pltpu.sync_copy(par_hbm, p_v)
    pltpu.sync_copy(z_v.at[pl.ds(0, RP)], deg_sp.at[pl.ds(wid * RP, RP)])
    pltpu.sync_copy(z_v, acc_sp.at[nsl])
    plsc.subcore_barrier()

    for c in range(NCHUNK):
        pltpu.sync_copy(dst_hbm.at[pl.ds(wid * EPW + c * EC, EC)], di_v)
        pltpu.sync_copy(one_v, deg_sp.at[di_v], add=True)
    plsc.subcore_barrier()

    pltpu.sync_copy(deg_sp.at[pl.ds(wid * RP, RP)], d_v)

    def _dinv(i, _):
        s = pl.ds(i * 16, 16)
        d_v[s] = _rsqrt16(d_v[s] + 1.0)
        return _
    lax.fori_loop(0, RP // 16, _dinv, None)

    pltpu.sync_copy(h1_hbm.at[nsl], g_v)

    def _scale(i, _):
        s = pl.ds(i * 16, 16)
        dv = plsc.load_gather(d_v, [(lanes + i * 16) >> 2])
        g_v[s] = g_v[s] * dv
        return _
    lax.fori_loop(0, RP4 // 16, _scale, None)

    pltpu.sync_copy(g_v, g_sp.at[nsl])
    plsc.subcore_barrier()

    def _edge_pass(srci_hbm, dsti_hbm, gi, si, msg, mult):
        for c in range(NCHUNK):
            esl = pl.ds((wid * EPW + c * EC) * mult, EC * mult)
            pltpu.sync_copy(srci_hbm.at[esl], gi)
            pltpu.async_copy(g_sp.at[gi], msg, sem).wait()
            pltpu.sync_copy(dsti_hbm.at[esl], si)
            pltpu.sync_copy(msg, acc_sp.at[si], add=True)

    for layer in range(3):
        if layer < 2:
            _edge_pass(src4_hbm, dst4_hbm, gi_v, si_v, msg_v, 4)
        else:
            _edge_pass(src2_hbm, dst2_hbm, gi2_v, si2_v, msg2_v, 2)
        plsc.subcore_barrier()

        pltpu.sync_copy(acc_sp.at[nsl], a_v)
        boff = 48 + 4 * layer

        def _node(i, _):
            s = pl.ds(i * 16, 16)
            flat = lanes + i * 16
            dv = plsc.load_gather(d_v, [flat >> 2])
            bv = plsc.load_gather(p_v, [boff + (flat & 3)])
            h_v[s] = _tanh16(dv * (a_v[s] + g_v[s]) + bv)
            return _
        lax.fori_loop(0, RP4 // 16, _node, None)

        if layer < 2:
            woff = 16 * layer

            def _mm(i, _):
                s = pl.ds(i * 16, 16)
                flat = lanes + i * 16
                nd4 = (flat >> 2) << 2
                cc = flat & 3
                acc = jnp.zeros((16,), jnp.float32)
                for k in range(4):
                    hk = plsc.load_gather(h_v, [nd4 + k])
                    wk = plsc.load_gather(p_v, [woff + 4 * k + cc])
                    acc = acc + hk * wk
                dv = plsc.load_gather(d_v, [flat >> 2])
                g_v[s] = acc * dv
                return _
            lax.fori_loop(0, RP4 // 16, _mm, None)

            pltpu.sync_copy(g_v, g_sp.at[nsl])
            pltpu.sync_copy(z_v, acc_sp.at[nsl])
        else:
            def _cls(i, _):
                s = pl.ds(i * 16, 16)
                flat = lanes + i * 16
                nd4 = (flat >> 2) << 2
                cc = flat & 3
                acc = plsc.load_gather(p_v, [60 + cc])
                for k in range(4):
                    hk = plsc.load_gather(h_v, [nd4 + k])
                    wk = plsc.load_gather(p_v, [32 + 4 * k + cc])
                    acc = acc + hk * wk
                a_v[s] = acc
                return _
            lax.fori_loop(0, RP4 // 16, _cls, None)

            pltpu.sync_copy(a_v, out_hbm.at[nsl])

            def _hout(i, _):
                s = pl.ds(i * 16, 16)
                f2 = lanes + i * 16
                g_v[s] = plsc.load_gather(h_v, [((f2 >> 1) << 2) + (f2 & 1)])
                return _
            lax.fori_loop(0, RP * 2 // 16, _hout, None)

            pltpu.sync_copy(g_v.at[pl.ds(0, RP * 2)],
                            hout_hbm.at[pl.ds(wid * RP * 2, RP * 2)])
        plsc.subcore_barrier()


def kernel(x, edge_index, W1, b1, W2, b2, W3, b3, Wc, bc):
    src = edge_index[0]
    dst = edge_index[1]

    h1 = pl.pallas_call(
        _mm_body,
        out_shape=jax.ShapeDtypeStruct((N, 4), jnp.float32),
    )(x, W1)

    h1f = jnp.pad(h1, ((0, NP - N), (0, 0))).ravel()
    four = jnp.arange(4, dtype=jnp.int32)
    two = jnp.arange(2, dtype=jnp.int32)
    src4 = (4 * src[:, None] + four).ravel()
    dst4 = (4 * dst[:, None] + four).ravel()
    src2 = (4 * src[:, None] + two).ravel()
    dst2 = (4 * dst[:, None] + two).ravel()
    w3p = jnp.pad(W3, ((0, 0), (0, 2)))
    wcp = jnp.pad(Wc, ((0, 2), (0, 0)))
    b3p = jnp.pad(b3, (0, 2))
    par = jnp.concatenate([W2.ravel(), w3p.ravel(), wcp.ravel(),
                           b1, b2, b3p, bc]).astype(jnp.float32)

    out_f, h_f = _gcn_sc(dst, src4, dst4, src2, dst2, h1f, par)
    out = out_f.reshape(NP, 4)[:N]
    h = h_f.reshape(NP, 2)[:N]
    return (out, h)


# speedup vs baseline: 1.5734x; 1.5660x over previous
import functools
import jax
import jax.numpy as jnp
from jax import lax
from jax.experimental import pallas as pl
from jax.experimental.pallas import tpu as pltpu
from jax.experimental.pallas import tpu_sc as plsc

N = 10000
E = 320000
NS = 16
NP = 10240
RP = NP // NS
RP4 = RP * 4
EPW = E // NS
EC = 4000
NCHUNK = EPW // EC

_mesh = plsc.VectorSubcoreMesh(core_axis_name="c", subcore_axis_name="s",
                               num_cores=1)


def _mm_body(x_ref, w_ref, o_ref):
    o_ref[...] = jnp.dot(x_ref[...], w_ref[...],
                         preferred_element_type=jnp.float32)


def _rsqrt16(x):
    i = plsc.bitcast(x, jnp.int32)
    y = plsc.bitcast(jnp.int32(0x5F3759DF) - (i >> 1), jnp.float32)
    for _ in range(4):
        y = y * (1.5 - 0.5 * x * y * y)
    return y


def _tanh16(x):
    e = jnp.exp(2.0 * x)
    return 1.0 - 2.0 / (e + 1.0)


@functools.partial(
    pl.kernel,
    out_type=[
        jax.ShapeDtypeStruct((NP * 4,), jnp.float32),
        jax.ShapeDtypeStruct((NP * 2,), jnp.float32),
    ],
    mesh=_mesh,
    compiler_params=pltpu.CompilerParams(needs_layout_passes=False),
    scratch_types=[
        pltpu.VMEM_SHARED((NP * 4,), jnp.float32),
        pltpu.VMEM_SHARED((NP * 4,), jnp.float32),
        pltpu.VMEM_SHARED((NP,), jnp.float32),
        pltpu.VMEM((EC * 4,), jnp.int32),
        pltpu.VMEM((EC * 4,), jnp.int32),
        pltpu.VMEM((EC * 4,), jnp.float32),
        pltpu.VMEM((EC,), jnp.int32),
        pltpu.VMEM((EC,), jnp.float32),
        pltpu.VMEM((RP4,), jnp.float32),
        pltpu.VMEM((RP4,), jnp.float32),
        pltpu.VMEM((RP4,), jnp.float32),
        pltpu.VMEM((RP4,), jnp.float32),
        pltpu.VMEM((RP,), jnp.float32),
        pltpu.VMEM((64,), jnp.float32),
        pltpu.SemaphoreType.DMA,
    ],
)
def _gcn_sc(dst_hbm, src4_hbm, dst4_hbm, h1_hbm, par_hbm,
            out_hbm, hout_hbm,
            g_sp, acc_sp, deg_sp, gi_v, si_v, msg_v,
            di_v, one_v, z_v, g_v, a_v, h_v, d_v, p_v, sem):
    wid = lax.axis_index("s")
    nsl = pl.ds(wid * RP4, RP4)
    lanes = lax.iota(jnp.int32, 16)

    def _fill(i, _):
        one_v[pl.ds(i * 16, 16)] = jnp.full((16,), 1.0, jnp.float32)
        return _
    lax.fori_loop(0, EC // 16, _fill, None)

    def _zfill(i, _):
        z_v[pl.ds(i * 16, 16)] = jnp.zeros((16,), jnp.float32)
        return _
    lax.fori_loop(0, RP4 // 16, _zfill, None)

    pltpu.sync_copy(par_hbm, p_v)
    pltpu.sync_copy(z_v.at[pl.ds(0, RP)], deg_sp.at[pl.ds(wid * RP, RP)])
    pltpu.sync_copy(z_v, acc_sp.at[nsl])
    plsc.subcore_barrier()

    for c in range(NCHUNK):
        pltpu.sync_copy(dst_hbm.at[pl.ds(wid * EPW + c * EC, EC)], di_v)
        pltpu.sync_copy(one_v, deg_sp.at[di_v], add=True)
    plsc.subcore_barrier()

    pltpu.sync_copy(deg_sp.at[pl.ds(wid * RP, RP)], d_v)

    def _dinv(i, _):
        s = pl.ds(i * 16, 16)
        d_v[s] = _rsqrt16(d_v[s] + 1.0)
        return _
    lax.fori_loop(0, RP // 16, _dinv, None)

    pltpu.sync_copy(h1_hbm.at[nsl], g_v)

    def _scale(i, _):
        s = pl.ds(i * 16, 16)
        dv = plsc.load_gather(d_v, [(lanes + i * 16) >> 2])
        g_v[s] = g_v[s] * dv
        return _
    lax.fori_loop(0, RP4 // 16, _scale, None)

    pltpu.sync_copy(g_v, g_sp.at[nsl])
    plsc.subcore_barrier()

    def _edge_pass(srci_hbm, dsti_hbm, gi, si, msg, mult):
        for c in range(NCHUNK):
            esl = pl.ds((wid * EPW + c * EC) * mult, EC * mult)
            pltpu.sync_copy(srci_hbm.at[esl], gi)
            pltpu.async_copy(g_sp.at[gi], msg, sem).wait()
            pltpu.sync_copy(dsti_hbm.at[esl], si)
            pltpu.sync_copy(msg, acc_sp.at[si], add=True)

    for layer in range(3):
        _edge_pass(src4_hbm, dst4_hbm, gi_v, si_v, msg_v, 4)
        plsc.subcore_barrier()

        pltpu.sync_copy(acc_sp.at[nsl], a_v)
        boff = 48 + 4 * layer

        def _node(i, _):
            s = pl.ds(i * 16, 16)
            flat = lanes + i * 16
            dv = plsc.load_gather(d_v, [flat >> 2])
            bv = plsc.load_gather(p_v, [boff + (flat & 3)])
            h_v[s] = _tanh16(dv * (a_v[s] + g_v[s]) + bv)
            return _
        lax.fori_loop(0, RP4 // 16, _node, None)

        if layer < 2:
            woff = 16 * layer

            def _mm(i, _):
                s = pl.ds(i * 16, 16)
                flat = lanes + i * 16
                nd4 = (flat >> 2) << 2
                cc = flat & 3
                acc = jnp.zeros((16,), jnp.float32)
                for k in range(4):
                    hk = plsc.load_gather(h_v, [nd4 + k])
                    wk = plsc.load_gather(p_v, [woff + 4 * k + cc])
                    acc = acc + hk * wk
                dv = plsc.load_gather(d_v, [flat >> 2])
                g_v[s] = acc * dv
                return _
            lax.fori_loop(0, RP4 // 16, _mm, None)

            pltpu.sync_copy(g_v, g_sp.at[nsl])
            pltpu.sync_copy(z_v, acc_sp.at[nsl])
        else:
            def _cls(i, _):
                s = pl.ds(i * 16, 16)
                flat = lanes + i * 16
                nd4 = (flat >> 2) << 2
                cc = flat & 3
                acc = plsc.load_gather(p_v, [60 + cc])
                for k in range(4):
                    hk = plsc.load_gather(h_v, [nd4 + k])
                    wk = plsc.load_gather(p_v, [32 + 4 * k + cc])
                    acc = acc + hk * wk
                a_v[s] = acc
                return _
            lax.fori_loop(0, RP4 // 16, _cls, None)

            pltpu.sync_copy(a_v, out_hbm.at[nsl])

            def _hout(i, _):
                s = pl.ds(i * 16, 16)
                f2 = lanes + i * 16
                g_v[s] = plsc.load_gather(h_v, [((f2 >> 1) << 2) + (f2 & 1)])
                return _
            lax.fori_loop(0, RP * 2 // 16, _hout, None)

            pltpu.sync_copy(g_v.at[pl.ds(0, RP * 2)],
                            hout_hbm.at[pl.ds(wid * RP * 2, RP * 2)])
        plsc.subcore_barrier()


def kernel(x, edge_index, W1, b1, W2, b2, W3, b3, Wc, bc):
    src = edge_index[0]
    dst = edge_index[1]

    h1 = pl.pallas_call(
        _mm_body,
        out_shape=jax.ShapeDtypeStruct((N, 4), jnp.float32),
    )(x, W1)

    h1f = jnp.pad(h1, ((0, NP - N), (0, 0))).ravel()
    four = jnp.arange(4, dtype=jnp.int32)
    src4 = (4 * src[:, None] + four).ravel()
    dst4 = (4 * dst[:, None] + four).ravel()
    w3p = jnp.pad(W3, ((0, 0), (0, 2)))
    wcp = jnp.pad(Wc, ((0, 2), (0, 0)))
    b3p = jnp.pad(b3, (0, 2))
    par = jnp.concatenate([W2.ravel(), w3p.ravel(), wcp.ravel(),
                           b1, b2, b3p, bc]).astype(jnp.float32)

    out_f, h_f = _gcn_sc(dst, src4, dst4, h1f, par)
    out = out_f.reshape(NP, 4)[:N]
    h = h_f.reshape(NP, 2)[:N]
    return (out, h)


# speedup vs baseline: 2.7916x; 1.7742x over previous
import functools
import jax
import jax.numpy as jnp
from jax import lax
from jax.experimental import pallas as pl
from jax.experimental.pallas import tpu as pltpu
from jax.experimental.pallas import tpu_sc as plsc

N = 10000
E = 320000
NS = 16
NP = 10240
RP = NP // NS
RP4 = RP * 4
EPW = E // NS
EC = 4000
NCHUNK = EPW // EC

_mesh = plsc.VectorSubcoreMesh(core_axis_name="c", subcore_axis_name="s",
                               num_cores=1)


def _mm_body(x_ref, w_ref, o_ref):
    o_ref[...] = jnp.dot(x_ref[...], w_ref[...],
                         preferred_element_type=jnp.float32)


def _rsqrt16(x):
    i = plsc.bitcast(x, jnp.int32)
    y = plsc.bitcast(jnp.int32(0x5F3759DF) - (i >> 1), jnp.float32)
    for _ in range(4):
        y = y * (1.5 - 0.5 * x * y * y)
    return y


def _tanh16(x):
    e = jnp.exp(2.0 * x)
    return 1.0 - 2.0 / (e + 1.0)


@functools.partial(
    pl.kernel,
    out_type=[
        jax.ShapeDtypeStruct((NP * 4,), jnp.float32),
        jax.ShapeDtypeStruct((NP * 2,), jnp.float32),
    ],
    mesh=_mesh,
    compiler_params=pltpu.CompilerParams(needs_layout_passes=False),
    scratch_types=[
        pltpu.VMEM_SHARED((NP * 4,), jnp.float32),
        pltpu.VMEM_SHARED((NP * 4,), jnp.float32),
        pltpu.VMEM_SHARED((NP,), jnp.float32),
        pltpu.VMEM((EC * 4,), jnp.int32),
        pltpu.VMEM((EC * 4,), jnp.int32),
        pltpu.VMEM((EC * 4,), jnp.float32),
        pltpu.VMEM((EC,), jnp.int32),
        pltpu.VMEM((EC,), jnp.int32),
        pltpu.VMEM((EC,), jnp.float32),
        pltpu.VMEM((RP4,), jnp.float32),
        pltpu.VMEM((RP4,), jnp.float32),
        pltpu.VMEM((RP4,), jnp.float32),
        pltpu.VMEM((RP4,), jnp.float32),
        pltpu.VMEM((RP,), jnp.float32),
        pltpu.VMEM((64,), jnp.float32),
        pltpu.SemaphoreType.DMA,
    ],
)
def _gcn_sc(dst_hbm, src_hbm, h1_hbm, par_hbm,
            out_hbm, hout_hbm,
            g_sp, acc_sp, deg_sp, gi_v, si_v, msg_v,
            di_v, ri_v, one_v, z_v, g_v, a_v, h_v, d_v, p_v, sem):
    wid = lax.axis_index("s")
    nsl = pl.ds(wid * RP4, RP4)
    lanes = lax.iota(jnp.int32, 16)

    def _fill(i, _):
        one_v[pl.ds(i * 16, 16)] = jnp.full((16,), 1.0, jnp.float32)
        return _
    lax.fori_loop(0, EC // 16, _fill, None)

    def _zfill(i, _):
        z_v[pl.ds(i * 16, 16)] = jnp.zeros((16,), jnp.float32)
        return _
    lax.fori_loop(0, RP4 // 16, _zfill, None)

    pltpu.sync_copy(par_hbm, p_v)
    pltpu.sync_copy(z_v.at[pl.ds(0, RP)], deg_sp.at[pl.ds(wid * RP, RP)])
    pltpu.sync_copy(z_v, acc_sp.at[nsl])
    plsc.subcore_barrier()

    for c in range(NCHUNK):
        pltpu.sync_copy(dst_hbm.at[pl.ds(wid * EPW + c * EC, EC)], di_v)
        pltpu.sync_copy(one_v, deg_sp.at[di_v], add=True)
    plsc.subcore_barrier()

    pltpu.sync_copy(deg_sp.at[pl.ds(wid * RP, RP)], d_v)

    def _dinv(i, _):
        s = pl.ds(i * 16, 16)
        d_v[s] = _rsqrt16(d_v[s] + 1.0)
        return _
    lax.fori_loop(0, RP // 16, _dinv, None)

    pltpu.sync_copy(h1_hbm.at[nsl], g_v)

    def _scale(i, _):
        s = pl.ds(i * 16, 16)
        dv = plsc.load_gather(d_v, [(lanes + i * 16) >> 2])
        g_v[s] = g_v[s] * dv
        return _
    lax.fori_loop(0, RP4 // 16, _scale, None)

    pltpu.sync_copy(g_v, g_sp.at[nsl])
    plsc.subcore_barrier()

    def _expand4(raw, out):
        def body(i, _):
            flat = lanes + i * 16
            rv = plsc.load_gather(raw, [flat >> 2])
            out[pl.ds(i * 16, 16)] = (rv << 2) + (flat & 3)
            return _
        lax.fori_loop(0, EC * 4 // 16, body, None)

    def _edge_pass(gi, si, msg):
        for c in range(NCHUNK):
            esl = pl.ds(wid * EPW + c * EC, EC)
            pltpu.sync_copy(src_hbm.at[esl], di_v)
            _expand4(di_v, gi)
            pltpu.async_copy(g_sp.at[gi], msg, sem).wait()
            pltpu.sync_copy(dst_hbm.at[esl], ri_v)
            _expand4(ri_v, si)
            pltpu.sync_copy(msg, acc_sp.at[si], add=True)

    for layer in range(3):
        _edge_pass(gi_v, si_v, msg_v)
        plsc.subcore_barrier()

        pltpu.sync_copy(acc_sp.at[nsl], a_v)
        boff = 48 + 4 * layer

        def _node(i, _):
            s = pl.ds(i * 16, 16)
            flat = lanes + i * 16
            dv = plsc.load_gather(d_v, [flat >> 2])
            bv = plsc.load_gather(p_v, [boff + (flat & 3)])
            h_v[s] = _tanh16(dv * (a_v[s] + g_v[s]) + bv)
            return _
        lax.fori_loop(0, RP4 // 16, _node, None)

        if layer < 2:
            woff = 16 * layer

            def _mm(i, _):
                s = pl.ds(i * 16, 16)
                flat = lanes + i * 16
                nd4 = (flat >> 2) << 2
                cc = flat & 3
                acc = jnp.zeros((16,), jnp.float32)
                for k in range(4):
                    hk = plsc.load_gather(h_v, [nd4 + k])
                    wk = plsc.load_gather(p_v, [woff + 4 * k + cc])
                    acc = acc + hk * wk
                dv = plsc.load_gather(d_v, [flat >> 2])
                g_v[s] = acc * dv
                return _
            lax.fori_loop(0, RP4 // 16, _mm, None)

            pltpu.sync_copy(g_v, g_sp.at[nsl])
            pltpu.sync_copy(z_v, acc_sp.at[nsl])
        else:
            def _cls(i, _):
                s = pl.ds(i * 16, 16)
                flat = lanes + i * 16
                nd4 = (flat >> 2) << 2
                cc = flat & 3
                acc = plsc.load_gather(p_v, [60 + cc])
                for k in range(4):
                    hk = plsc.load_gather(h_v, [nd4 + k])
                    wk = plsc.load_gather(p_v, [32 + 4 * k + cc])
                    acc = acc + hk * wk
                a_v[s] = acc
                return _
            lax.fori_loop(0, RP4 // 16, _cls, None)

            pltpu.sync_copy(a_v, out_hbm.at[nsl])

            def _hout(i, _):
                s = pl.ds(i * 16, 16)
                f2 = lanes + i * 16
                g_v[s] = plsc.load_gather(h_v, [((f2 >> 1) << 2) + (f2 & 1)])
                return _
            lax.fori_loop(0, RP * 2 // 16, _hout, None)

            pltpu.sync_copy(g_v.at[pl.ds(0, RP * 2)],
                            hout_hbm.at[pl.ds(wid * RP * 2, RP * 2)])
        plsc.subcore_barrier()


def kernel(x, edge_index, W1, b1, W2, b2, W3, b3, Wc, bc):
    src = edge_index[0]
    dst = edge_index[1]

    h1 = pl.pallas_call(
        _mm_body,
        out_shape=jax.ShapeDtypeStruct((N, 4), jnp.float32),
    )(x, W1)

    h1f = jnp.pad(h1, ((0, NP - N), (0, 0))).ravel()
    w3p = jnp.pad(W3, ((0, 0), (0, 2)))
    wcp = jnp.pad(Wc, ((0, 2), (0, 0)))
    b3p = jnp.pad(b3, (0, 2))
    par = jnp.concatenate([W2.ravel(), w3p.ravel(), wcp.ravel(),
                           b1, b2, b3p, bc]).astype(jnp.float32)

    out_f, h_f = _gcn_sc(dst, src, h1f, par)
    out = out_f.reshape(NP, 4)[:N]
    h = h_f.reshape(NP, 2)[:N]
    return (out, h)


# speedup vs baseline: 2.8543x; 1.0225x over previous
import functools
import jax
import jax.numpy as jnp
from jax import lax
from jax.experimental import pallas as pl
from jax.experimental.pallas import tpu as pltpu
from jax.experimental.pallas import tpu_sc as plsc

N = 10000
E = 320000
NS = 16
NP = 10240
RP = NP // NS
RP4 = RP * 4
EPW = E // NS
EC = 4000
NCHUNK = EPW // EC

_mesh = plsc.VectorSubcoreMesh(core_axis_name="c", subcore_axis_name="s",
                               num_cores=1)


def _mm_body(x_ref, w_ref, o_ref):
    o_ref[...] = jnp.dot(x_ref[...], w_ref[...],
                         preferred_element_type=jnp.float32)


def _rsqrt16(x):
    i = plsc.bitcast(x, jnp.int32)
    y = plsc.bitcast(jnp.int32(0x5F3759DF) - (i >> 1), jnp.float32)
    for _ in range(4):
        y = y * (1.5 - 0.5 * x * y * y)
    return y


def _tanh16(x):
    e = jnp.exp(2.0 * x)
    return 1.0 - 2.0 / (e + 1.0)


@functools.partial(
    pl.kernel,
    out_type=[
        jax.ShapeDtypeStruct((NP * 4,), jnp.float32),
        jax.ShapeDtypeStruct((NP * 2,), jnp.float32),
    ],
    mesh=_mesh,
    compiler_params=pltpu.CompilerParams(needs_layout_passes=False),
    scratch_types=[
        pltpu.VMEM_SHARED((NP * 4,), jnp.float32),
        pltpu.VMEM_SHARED((NP * 4,), jnp.float32),
        pltpu.VMEM_SHARED((NP,), jnp.float32),
        [pltpu.VMEM((EC * 4,), jnp.int32)] * 2,
        [pltpu.VMEM((EC * 4,), jnp.int32)] * 2,
        [pltpu.VMEM((EC * 4,), jnp.float32)] * 2,
        pltpu.VMEM((EC,), jnp.int32),
        pltpu.VMEM((EC,), jnp.int32),
        pltpu.VMEM((EC,), jnp.float32),
        pltpu.VMEM((RP4,), jnp.float32),
        pltpu.VMEM((RP4,), jnp.float32),
        pltpu.VMEM((RP4,), jnp.float32),
        pltpu.VMEM((RP4,), jnp.float32),
        pltpu.VMEM((RP,), jnp.float32),
        pltpu.VMEM((64,), jnp.float32),
        [pltpu.SemaphoreType.DMA] * 2,
        pltpu.SemaphoreType.DMA,
    ],
)
def _gcn_sc(dst_hbm, src_hbm, h1_hbm, par_hbm,
            out_hbm, hout_hbm,
            g_sp, acc_sp, deg_sp, gi_v, si_v, msg_v,
            di_v, ri_v, one_v, z_v, g_v, a_v, h_v, d_v, p_v, gsems, sem):
    wid = lax.axis_index("s")
    nsl = pl.ds(wid * RP4, RP4)
    lanes = lax.iota(jnp.int32, 16)

    def _fill(i, _):
        one_v[pl.ds(i * 16, 16)] = jnp.full((16,), 1.0, jnp.float32)
        return _
    lax.fori_loop(0, EC // 16, _fill, None)

    def _zfill(i, _):
        z_v[pl.ds(i * 16, 16)] = jnp.zeros((16,), jnp.float32)
        return _
    lax.fori_loop(0, RP4 // 16, _zfill, None)

    pltpu.sync_copy(par_hbm, p_v)
    pltpu.sync_copy(z_v.at[pl.ds(0, RP)], deg_sp.at[pl.ds(wid * RP, RP)])
    pltpu.sync_copy(z_v, acc_sp.at[nsl])
    plsc.subcore_barrier()

    for c in range(NCHUNK):
        pltpu.sync_copy(dst_hbm.at[pl.ds(wid * EPW + c * EC, EC)], di_v)
        pltpu.sync_copy(one_v, deg_sp.at[di_v], add=True)
    plsc.subcore_barrier()

    pltpu.sync_copy(deg_sp.at[pl.ds(wid * RP, RP)], d_v)

    def _dinv(i, _):
        s = pl.ds(i * 16, 16)
        d_v[s] = _rsqrt16(d_v[s] + 1.0)
        return _
    lax.fori_loop(0, RP // 16, _dinv, None)

    pltpu.sync_copy(h1_hbm.at[nsl], g_v)

    def _scale(i, _):
        s = pl.ds(i * 16, 16)
        dv = plsc.load_gather(d_v, [(lanes + i * 16) >> 2])
        g_v[s] = g_v[s] * dv
        return _
    lax.fori_loop(0, RP4 // 16, _scale, None)

    pltpu.sync_copy(g_v, g_sp.at[nsl])
    plsc.subcore_barrier()

    l4 = lanes >> 2
    lm = lanes & 3

    def _expand4(raw, out):
        def body(i, _):
            rv = plsc.load_gather(raw, [l4 + i * 4])
            out[pl.ds(i * 16, 16)] = (rv << 2) + lm
            return _
        lax.fori_loop(0, EC * 4 // 16, body, None)

    def _stage(c, gi, si):
        esl = pl.ds(wid * EPW + c * EC, EC)
        pltpu.sync_copy(src_hbm.at[esl], di_v)
        _expand4(di_v, gi)
        pltpu.sync_copy(dst_hbm.at[esl], ri_v)
        _expand4(ri_v, si)

    def _edge_pass(gi, si, msg):
        _stage(0, gi[0], si[0])
        gd = pltpu.async_copy(g_sp.at[gi[0]], msg[0], gsems[0])
        for c in range(NCHUNK):
            b = c % 2
            nb = 1 - b
            if c + 1 < NCHUNK:
                _stage(c + 1, gi[nb], si[nb])
            gd.wait()
            if c + 1 < NCHUNK:
                gd = pltpu.async_copy(g_sp.at[gi[nb]], msg[nb], gsems[nb])
            pltpu.sync_copy(msg[b], acc_sp.at[si[b]], add=True)

    for layer in range(3):
        _edge_pass(gi_v, si_v, msg_v)
        plsc.subcore_barrier()

        pltpu.sync_copy(acc_sp.at[nsl], a_v)
        boff = 48 + 4 * layer

        def _node(i, _):
            s = pl.ds(i * 16, 16)
            flat = lanes + i * 16
            dv = plsc.load_gather(d_v, [flat >> 2])
            bv = plsc.load_gather(p_v, [boff + (flat & 3)])
            h_v[s] = _tanh16(dv * (a_v[s] + g_v[s]) + bv)
            return _
        lax.fori_loop(0, RP4 // 16, _node, None)

        if layer < 2:
            woff = 16 * layer

            def _mm(i, _):
                s = pl.ds(i * 16, 16)
                flat = lanes + i * 16
                nd4 = (flat >> 2) << 2
                cc = flat & 3
                acc = jnp.zeros((16,), jnp.float32)
                for k in range(4):
                    hk = plsc.load_gather(h_v, [nd4 + k])
                    wk = plsc.load_gather(p_v, [woff + 4 * k + cc])
                    acc = acc + hk * wk
                dv = plsc.load_gather(d_v, [flat >> 2])
                g_v[s] = acc * dv
                return _
            lax.fori_loop(0, RP4 // 16, _mm, None)

            pltpu.sync_copy(g_v, g_sp.at[nsl])
            pltpu.sync_copy(z_v, acc_sp.at[nsl])
        else:
            def _cls(i, _):
                s = pl.ds(i * 16, 16)
                flat = lanes + i * 16
                nd4 = (flat >> 2) << 2
                cc = flat & 3
                acc = plsc.load_gather(p_v, [60 + cc])
                for k in range(4):
                    hk = plsc.load_gather(h_v, [nd4 + k])
                    wk = plsc.load_gather(p_v, [32 + 4 * k + cc])
                    acc = acc + hk * wk
                a_v[s] = acc
                return _
            lax.fori_loop(0, RP4 // 16, _cls, None)

            pltpu.sync_copy(a_v, out_hbm.at[nsl])

            def _hout(i, _):
                s = pl.ds(i * 16, 16)
                f2 = lanes + i * 16
                g_v[s] = plsc.load_gather(h_v, [((f2 >> 1) << 2) + (f2 & 1)])
                return _
            lax.fori_loop(0, RP * 2 // 16, _hout, None)

            pltpu.sync_copy(g_v.at[pl.ds(0, RP * 2)],
                            hout_hbm.at[pl.ds(wid * RP * 2, RP * 2)])
        plsc.subcore_barrier()


def kernel(x, edge_index, W1, b1, W2, b2, W3, b3, Wc, bc):
    src = edge_index[0]
    dst = edge_index[1]

    h1 = pl.pallas_call(
        _mm_body,
        out_shape=jax.ShapeDtypeStruct((N, 4), jnp.float32),
    )(x, W1)

    h1f = jnp.pad(h1, ((0, NP - N), (0, 0))).ravel()
    w3p = jnp.pad(W3, ((0, 0), (0, 2)))
    wcp = jnp.pad(Wc, ((0, 2), (0, 0)))
    b3p = jnp.pad(b3, (0, 2))
    par = jnp.concatenate([W2.ravel(), w3p.ravel(), wcp.ravel(),
                           b1, b2, b3p, bc]).astype(jnp.float32)

    out_f, h_f = _gcn_sc(dst, src, h1f, par)
    out = out_f.reshape(NP, 4)[:N]
    h = h_f.reshape(NP, 2)[:N]
    return (out, h)


# speedup vs baseline: 3.4703x; 1.2158x over previous
import functools
import jax
import jax.numpy as jnp
from jax import lax
from jax.experimental import pallas as pl
from jax.experimental.pallas import tpu as pltpu
from jax.experimental.pallas import tpu_sc as plsc

N = 10000
E = 320000
NS = 16
NP = 10240
RP = NP // NS
RP4 = RP * 4
EPW = E // NS
EC = 2000
NCHUNK = EPW // EC

_mesh = plsc.VectorSubcoreMesh(core_axis_name="c", subcore_axis_name="s",
                               num_cores=1)


def _mm_body(x_ref, w_ref, o_ref):
    o_ref[...] = jnp.dot(x_ref[...], w_ref[...],
                         preferred_element_type=jnp.float32)


def _rsqrt16(x):
    i = plsc.bitcast(x, jnp.int32)
    y = plsc.bitcast(jnp.int32(0x5F3759DF) - (i >> 1), jnp.float32)
    for _ in range(4):
        y = y * (1.5 - 0.5 * x * y * y)
    return y


def _tanh16(x):
    e = jnp.exp(2.0 * x)
    return 1.0 - 2.0 / (e + 1.0)


@functools.partial(
    pl.kernel,
    out_type=[
        jax.ShapeDtypeStruct((NP * 4,), jnp.float32),
        jax.ShapeDtypeStruct((NP * 2,), jnp.float32),
    ],
    mesh=_mesh,
    compiler_params=pltpu.CompilerParams(needs_layout_passes=False),
    scratch_types=[
        pltpu.VMEM_SHARED((NP * 4,), jnp.float32),
        pltpu.VMEM_SHARED((NP * 4,), jnp.float32),
        pltpu.VMEM_SHARED((NP,), jnp.float32),
        pltpu.VMEM((NP * 4,), jnp.float32),
        [pltpu.VMEM((EC * 4,), jnp.int32)] * 2,
        [pltpu.VMEM((EC * 4,), jnp.float32)] * 2,
        [pltpu.VMEM((EC * 2,), jnp.int32)] * 2,
        [pltpu.VMEM((EC * 2,), jnp.float32)] * 2,
        pltpu.VMEM((EC,), jnp.int32),
        pltpu.VMEM((EC,), jnp.int32),
        pltpu.VMEM((EC,), jnp.float32),
        pltpu.VMEM((RP4,), jnp.float32),
        pltpu.VMEM((RP4,), jnp.float32),
        pltpu.VMEM((RP4,), jnp.float32),
        pltpu.VMEM((RP4,), jnp.float32),
        pltpu.VMEM((RP,), jnp.float32),
        pltpu.VMEM((64,), jnp.float32),
        [pltpu.SemaphoreType.DMA] * 2,
        pltpu.SemaphoreType.DMA,
    ],
)
def _gcn_sc(dst_hbm, src_hbm, h1_hbm, par_hbm,
            out_hbm, hout_hbm,
            g_sp, acc_sp, deg_sp, gt_v, si_v, msg_v, si2_v, msg2_v,
            di_v, ri_v, one_v, z_v, g_v, a_v, h_v, d_v, p_v, gsems, sem):
    wid = lax.axis_index("s")
    nsl = pl.ds(wid * RP4, RP4)
    lanes = lax.iota(jnp.int32, 16)

    def _fill(i, _):
        one_v[pl.ds(i * 16, 16)] = jnp.full((16,), 1.0, jnp.float32)
        return _
    lax.fori_loop(0, EC // 16, _fill, None)

    def _zfill(i, _):
        z_v[pl.ds(i * 16, 16)] = jnp.zeros((16,), jnp.float32)
        return _
    lax.fori_loop(0, RP4 // 16, _zfill, None)

    pltpu.sync_copy(par_hbm, p_v)
    pltpu.sync_copy(z_v.at[pl.ds(0, RP)], deg_sp.at[pl.ds(wid * RP, RP)])
    pltpu.sync_copy(z_v, acc_sp.at[nsl])
    plsc.subcore_barrier()

    for c in range(NCHUNK):
        pltpu.sync_copy(dst_hbm.at[pl.ds(wid * EPW + c * EC, EC)], di_v)
        pltpu.sync_copy(one_v, deg_sp.at[di_v], add=True)
    plsc.subcore_barrier()

    pltpu.sync_copy(deg_sp.at[pl.ds(wid * RP, RP)], d_v)

    def _dinv(i, _):
        s = pl.ds(i * 16, 16)
        d_v[s] = _rsqrt16(d_v[s] + 1.0)
        return _
    lax.fori_loop(0, RP // 16, _dinv, None)

    pltpu.sync_copy(h1_hbm.at[nsl], g_v)

    def _scale(i, _):
        s = pl.ds(i * 16, 16)
        dv = plsc.load_gather(d_v, [(lanes + i * 16) >> 2])
        g_v[s] = g_v[s] * dv
        return _
    lax.fori_loop(0, RP4 // 16, _scale, None)

    pltpu.sync_copy(g_v, g_sp.at[nsl])
    plsc.subcore_barrier()

    def _edge_pass(si, msg, mult, lidx, lmod):
        iters = EC * mult // 16
        step = 16 // mult
        descs = [None, None]
        for c in range(NCHUNK):
            b = c % 2
            if descs[b] is not None:
                descs[b].wait()
            esl = pl.ds(wid * EPW + c * EC, EC)
            pltpu.sync_copy(src_hbm.at[esl], di_v)
            pltpu.sync_copy(dst_hbm.at[esl], ri_v)
            sib = si[b]
            msgb = msg[b]

            def body(i, _):
                s = pl.ds(i * 16, 16)
                rs = plsc.load_gather(di_v, [lidx + i * step])
                msgb[s] = plsc.load_gather(gt_v, [(rs << 2) + lmod])
                rd = plsc.load_gather(ri_v, [lidx + i * step])
                sib[s] = (rd << 2) + lmod
                return _
            lax.fori_loop(0, iters, body, None)
            descs[b] = pltpu.async_copy(msgb, acc_sp.at[sib], gsems[b],
                                        add=True)
        for d in descs:
            if d is not None:
                d.wait()

    l4 = lanes >> 2
    lm = lanes & 3
    l2 = lanes >> 1
    lm2 = lanes & 1

    for layer in range(3):
        pltpu.sync_copy(g_sp, gt_v)
        if layer < 2:
            _edge_pass(si_v, msg_v, 4, l4, lm)
        else:
            _edge_pass(si2_v, msg2_v, 2, l2, lm2)
        plsc.subcore_barrier()

        pltpu.sync_copy(acc_sp.at[nsl], a_v)
        boff = 48 + 4 * layer

        def _node(i, _):
            s = pl.ds(i * 16, 16)
            flat = lanes + i * 16
            dv = plsc.load_gather(d_v, [flat >> 2])
            bv = plsc.load_gather(p_v, [boff + (flat & 3)])
            h_v[s] = _tanh16(dv * (a_v[s] + g_v[s]) + bv)
            return _
        lax.fori_loop(0, RP4 // 16, _node, None)

        if layer < 2:
            woff = 16 * layer

            def _mm(i, _):
                s = pl.ds(i * 16, 16)
                flat = lanes + i * 16
                nd4 = (flat >> 2) << 2
                cc = flat & 3
                acc = jnp.zeros((16,), jnp.float32)
                for k in range(4):
                    hk = plsc.load_gather(h_v, [nd4 + k])
                    wk = plsc.load_gather(p_v, [woff + 4 * k + cc])
                    acc = acc + hk * wk
                dv = plsc.load_gather(d_v, [flat >> 2])
                g_v[s] = acc * dv
                return _
            lax.fori_loop(0, RP4 // 16, _mm, None)

            pltpu.sync_copy(g_v, g_sp.at[nsl])
            pltpu.sync_copy(z_v, acc_sp.at[nsl])
        else:
            def _cls(i, _):
                s = pl.ds(i * 16, 16)
                flat = lanes + i * 16
                nd4 = (flat >> 2) << 2
                cc = flat & 3
                acc = plsc.load_gather(p_v, [60 + cc])
                for k in range(4):
                    hk = plsc.load_gather(h_v, [nd4 + k])
                    wk = plsc.load_gather(p_v, [32 + 4 * k + cc])
                    acc = acc + hk * wk
                a_v[s] = acc
                return _
            lax.fori_loop(0, RP4 // 16, _cls, None)

            pltpu.sync_copy(a_v, out_hbm.at[nsl])

            def _hout(i, _):
                s = pl.ds(i * 16, 16)
                f2 = lanes + i * 16
                g_v[s] = plsc.load_gather(h_v, [((f2 >> 1) << 2) + (f2 & 1)])
                return _
            lax.fori_loop(0, RP * 2 // 16, _hout, None)

            pltpu.sync_copy(g_v.at[pl.ds(0, RP * 2)],
                            hout_hbm.at[pl.ds(wid * RP * 2, RP * 2)])
        plsc.subcore_barrier()


def kernel(x, edge_index, W1, b1, W2, b2, W3, b3, Wc, bc):
    src = edge_index[0]
    dst = edge_index[1]

    h1 = pl.pallas_call(
        _mm_body,
        out_shape=jax.ShapeDtypeStruct((N, 4), jnp.float32),
    )(x, W1)

    h1f = jnp.pad(h1, ((0, NP - N), (0, 0))).ravel()
    w3p = jnp.pad(W3, ((0, 0), (0, 2)))
    wcp = jnp.pad(Wc, ((0, 2), (0, 0)))
    b3p = jnp.pad(b3, (0, 2))
    par = jnp.concatenate([W2.ravel(), w3p.ravel(), wcp.ravel(),
                           b1, b2, b3p, bc]).astype(jnp.float32)

    out_f, h_f = _gcn_sc(dst, src, h1f, par)
    out = out_f.reshape(NP, 4)[:N]
    h = h_f.reshape(NP, 2)[:N]
    return (out, h)


# speedup vs baseline: 3.8442x; 1.1077x over previous
import functools
import jax
import jax.numpy as jnp
from jax import lax
from jax.experimental import pallas as pl
from jax.experimental.pallas import tpu as pltpu
from jax.experimental.pallas import tpu_sc as plsc

N = 10000
E = 320000
NS = 16
NP = 10240
RP = NP // NS
RP4 = RP * 4
EPW = E // NS
EC = 2000
NCHUNK = EPW // EC

_mesh = plsc.VectorSubcoreMesh(core_axis_name="c", subcore_axis_name="s",
                               num_cores=1)


def _mm_body(x_ref, w_ref, o_ref):
    o_ref[...] = jnp.dot(x_ref[...], w_ref[...],
                         preferred_element_type=jnp.float32)


def _rsqrt16(x):
    i = plsc.bitcast(x, jnp.int32)
    y = plsc.bitcast(jnp.int32(0x5F3759DF) - (i >> 1), jnp.float32)
    for _ in range(4):
        y = y * (1.5 - 0.5 * x * y * y)
    return y


def _tanh16(x):
    e = jnp.exp(2.0 * x)
    return 1.0 - 2.0 / (e + 1.0)


@functools.partial(
    pl.kernel,
    out_type=[
        jax.ShapeDtypeStruct((NP * 4,), jnp.float32),
        jax.ShapeDtypeStruct((NP * 2,), jnp.float32),
    ],
    mesh=_mesh,
    compiler_params=pltpu.CompilerParams(needs_layout_passes=False),
    scratch_types=[
        pltpu.VMEM_SHARED((NP * 4,), jnp.float32),
        pltpu.VMEM_SHARED((NP * 4,), jnp.float32),
        pltpu.VMEM_SHARED((NP,), jnp.float32),
        pltpu.VMEM((NP * 4,), jnp.float32),
        [pltpu.VMEM((EC * 4,), jnp.int32)] * 2,
        [pltpu.VMEM((EC * 4,), jnp.float32)] * 2,
        [pltpu.VMEM((EC * 2,), jnp.int32)] * 2,
        [pltpu.VMEM((EC * 2,), jnp.float32)] * 2,
        [pltpu.VMEM((EC,), jnp.int32)] * 2,
        [pltpu.VMEM((EC,), jnp.int32)] * 2,
        pltpu.VMEM((EC,), jnp.float32),
        pltpu.VMEM((RP4,), jnp.float32),
        pltpu.VMEM((RP4,), jnp.float32),
        pltpu.VMEM((RP4,), jnp.float32),
        pltpu.VMEM((RP4,), jnp.float32),
        pltpu.VMEM((RP,), jnp.float32),
        pltpu.VMEM((64,), jnp.float32),
        [pltpu.SemaphoreType.DMA] * 2,
        [pltpu.SemaphoreType.DMA] * 2,
        [pltpu.SemaphoreType.DMA] * 2,
        pltpu.SemaphoreType.DMA,
    ],
)
def _gcn_sc(dst_hbm, src_hbm, h1_hbm, par_hbm,
            out_hbm, hout_hbm,
            g_sp, acc_sp, deg_sp, gt_v, si_v, msg_v, si2_v, msg2_v,
            di_v, ri_v, one_v, z_v, g_v, a_v, h_v, d_v, p_v, gsems, rsems, tsems, sem):
    wid = lax.axis_index("s")
    nsl = pl.ds(wid * RP4, RP4)
    lanes = lax.iota(jnp.int32, 16)

    def _fill(i, _):
        one_v[pl.ds(i * 16, 16)] = jnp.full((16,), 1.0, jnp.float32)
        return _
    lax.fori_loop(0, EC // 16, _fill, None)

    def _zfill(i, _):
        z_v[pl.ds(i * 16, 16)] = jnp.zeros((16,), jnp.float32)
        return _
    lax.fori_loop(0, RP4 // 16, _zfill, None)

    pltpu.sync_copy(par_hbm, p_v)
    pltpu.sync_copy(z_v.at[pl.ds(0, RP)], deg_sp.at[pl.ds(wid * RP, RP)])
    pltpu.sync_copy(z_v, acc_sp.at[nsl])
    plsc.subcore_barrier()

    for c in range(NCHUNK):
        pltpu.sync_copy(dst_hbm.at[pl.ds(wid * EPW + c * EC, EC)], di_v[0])
        pltpu.sync_copy(one_v, deg_sp.at[di_v[0]], add=True)
    plsc.subcore_barrier()

    pltpu.sync_copy(deg_sp.at[pl.ds(wid * RP, RP)], d_v)

    def _dinv(i, _):
        s = pl.ds(i * 16, 16)
        d_v[s] = _rsqrt16(d_v[s] + 1.0)
        return _
    lax.fori_loop(0, RP // 16, _dinv, None)

    pltpu.sync_copy(h1_hbm.at[nsl], g_v)

    def _scale(i, _):
        s = pl.ds(i * 16, 16)
        dv = plsc.load_gather(d_v, [(lanes + i * 16) >> 2])
        g_v[s] = g_v[s] * dv
        return _
    lax.fori_loop(0, RP4 // 16, _scale, None)

    pltpu.sync_copy(g_v, g_sp.at[nsl])
    plsc.subcore_barrier()

    def _edge_pass(si, msg, mult, lidx, lmod):
        iters = EC * mult // 16
        step = 16 // mult

        def esl(c):
            return pl.ds(wid * EPW + c * EC, EC)

        descs = [None, None]
        sdesc = [pltpu.async_copy(src_hbm.at[esl(0)], di_v[0], rsems[0])]
        ddesc = [pltpu.async_copy(dst_hbm.at[esl(0)], ri_v[0], tsems[0])]
        for c in range(NCHUNK):
            b = c % 2
            nb = 1 - b
            if c + 1 < NCHUNK:
                sdesc.append(
                    pltpu.async_copy(src_hbm.at[esl(c + 1)], di_v[nb],
                                     rsems[nb]))
                ddesc.append(
                    pltpu.async_copy(dst_hbm.at[esl(c + 1)], ri_v[nb],
                                     tsems[nb]))
            if descs[b] is not None:
                descs[b].wait()
            sdesc[c].wait()
            ddesc[c].wait()
            sib = si[b]
            msgb = msg[b]
            dib = di_v[b]
            rib = ri_v[b]

            def body(i, _):
                s = pl.ds(i * 16, 16)
                rs = plsc.load_gather(dib, [lidx + i * step])
                msgb[s] = plsc.load_gather(gt_v, [(rs << 2) + lmod])
                rd = plsc.load_gather(rib, [lidx + i * step])
                sib[s] = (rd << 2) + lmod
                return _
            lax.fori_loop(0, iters, body, None)
            descs[b] = pltpu.async_copy(msgb, acc_sp.at[sib], gsems[b],
                                        add=True)
        for d in descs:
            if d is not None:
                d.wait()

    l4 = lanes >> 2
    lm = lanes & 3
    l2 = lanes >> 1
    lm2 = lanes & 1

    for layer in range(3):
        pltpu.sync_copy(g_sp, gt_v)
        if layer < 2:
            _edge_pass(si_v, msg_v, 4, l4, lm)
        else:
            _edge_pass(si2_v, msg2_v, 2, l2, lm2)
        plsc.subcore_barrier()

        pltpu.sync_copy(acc_sp.at[nsl], a_v)
        boff = 48 + 4 * layer

        def _node(i, _):
            s = pl.ds(i * 16, 16)
            flat = lanes + i * 16
            dv = plsc.load_gather(d_v, [flat >> 2])
            bv = plsc.load_gather(p_v, [boff + (flat & 3)])
            h_v[s] = _tanh16(dv * (a_v[s] + g_v[s]) + bv)
            return _
        lax.fori_loop(0, RP4 // 16, _node, None)

        if layer < 2:
            woff = 16 * layer

            def _mm(i, _):
                s = pl.ds(i * 16, 16)
                flat = lanes + i * 16
                nd4 = (flat >> 2) << 2
                cc = flat & 3
                acc = jnp.zeros((16,), jnp.float32)
                for k in range(4):
                    hk = plsc.load_gather(h_v, [nd4 + k])
                    wk = plsc.load_gather(p_v, [woff + 4 * k + cc])
                    acc = acc + hk * wk
                dv = plsc.load_gather(d_v, [flat >> 2])
                g_v[s] = acc * dv
                return _
            lax.fori_loop(0, RP4 // 16, _mm, None)

            pltpu.sync_copy(g_v, g_sp.at[nsl])
            pltpu.sync_copy(z_v, acc_sp.at[nsl])
        else:
            def _cls(i, _):
                s = pl.ds(i * 16, 16)
                flat = lanes + i * 16
                nd4 = (flat >> 2) << 2
                cc = flat & 3
                acc = plsc.load_gather(p_v, [60 + cc])
                for k in range(4):
                    hk = plsc.load_gather(h_v, [nd4 + k])
                    wk = plsc.load_gather(p_v, [32 + 4 * k + cc])
                    acc = acc + hk * wk
                a_v[s] = acc
                return _
            lax.fori_loop(0, RP4 // 16, _cls, None)

            pltpu.sync_copy(a_v, out_hbm.at[nsl])

            def _hout(i, _):
                s = pl.ds(i * 16, 16)
                f2 = lanes + i * 16
                g_v[s] = plsc.load_gather(h_v, [((f2 >> 1) << 2) + (f2 & 1)])
                return _
            lax.fori_loop(0, RP * 2 // 16, _hout, None)

            pltpu.sync_copy(g_v.at[pl.ds(0, RP * 2)],
                            hout_hbm.at[pl.ds(wid * RP * 2, RP * 2)])
        plsc.subcore_barrier()


def kernel(x, edge_index, W1, b1, W2, b2, W3, b3, Wc, bc):
    src = edge_index[0]
    dst = edge_index[1]

    h1 = pl.pallas_call(
        _mm_body,
        out_shape=jax.ShapeDtypeStruct((N, 4), jnp.float32),
    )(x, W1)

    h1f = jnp.pad(h1, ((0, NP - N), (0, 0))).ravel()
    w3p = jnp.pad(W3, ((0, 0), (0, 2)))
    wcp = jnp.pad(Wc, ((0, 2), (0, 0)))
    b3p = jnp.pad(b3, (0, 2))
    par = jnp.concatenate([W2.ravel(), w3p.ravel(), wcp.ravel(),
                           b1, b2, b3p, bc]).astype(jnp.float32)

    out_f, h_f = _gcn_sc(dst, src, h1f, par)
    out = out_f.reshape(NP, 4)[:N]
    h = h_f.reshape(NP, 2)[:N]
    return (out, h)


# speedup vs baseline: 3.9141x; 1.0182x over previous
import functools
import jax
import jax.numpy as jnp
from jax import lax
from jax.experimental import pallas as pl
from jax.experimental.pallas import tpu as pltpu
from jax.experimental.pallas import tpu_sc as plsc

N = 10000
E = 320000
NS = 16
NP = 10240
RP = NP // NS
RP4 = RP * 4
EPW = E // NS
EC = 2000
NCHUNK = EPW // EC

_mesh = plsc.VectorSubcoreMesh(core_axis_name="c", subcore_axis_name="s",
                               num_cores=1)


def _mm_body(x_ref, w_ref, o_ref):
    o_ref[...] = jnp.dot(x_ref[...], w_ref[...],
                         preferred_element_type=jnp.float32)


def _rsqrt16(x):
    i = plsc.bitcast(x, jnp.int32)
    y = plsc.bitcast(jnp.int32(0x5F3759DF) - (i >> 1), jnp.float32)
    for _ in range(4):
        y = y * (1.5 - 0.5 * x * y * y)
    return y


def _tanh16(x):
    e = jnp.exp(2.0 * x)
    return 1.0 - 2.0 / (e + 1.0)


@functools.partial(
    pl.kernel,
    out_type=[
        jax.ShapeDtypeStruct((NP * 4,), jnp.float32),
        jax.ShapeDtypeStruct((NP * 2,), jnp.float32),
    ],
    mesh=_mesh,
    compiler_params=pltpu.CompilerParams(needs_layout_passes=False),
    scratch_types=[
        pltpu.VMEM_SHARED((NP * 4,), jnp.float32),
        pltpu.VMEM_SHARED((NP * 4,), jnp.float32),
        pltpu.VMEM_SHARED((NP,), jnp.float32),
        pltpu.VMEM((NP * 4,), jnp.float32),
        [pltpu.VMEM((EC * 4,), jnp.int32)] * 2,
        [pltpu.VMEM((EC * 4,), jnp.float32)] * 2,
        [pltpu.VMEM((EC * 2,), jnp.int32)] * 2,
        [pltpu.VMEM((EC * 2,), jnp.float32)] * 2,
        [pltpu.VMEM((EC,), jnp.int32)] * 2,
        [pltpu.VMEM((EC,), jnp.int32)] * 2,
        pltpu.VMEM((EC,), jnp.float32),
        pltpu.VMEM((RP4,), jnp.float32),
        pltpu.VMEM((RP4,), jnp.float32),
        pltpu.VMEM((RP4,), jnp.float32),
        pltpu.VMEM((RP4,), jnp.float32),
        pltpu.VMEM((RP,), jnp.float32),
        pltpu.VMEM((64,), jnp.float32),
        [pltpu.SemaphoreType.DMA] * 2,
        [pltpu.SemaphoreType.DMA] * 2,
        [pltpu.SemaphoreType.DMA] * 2,
        pltpu.SemaphoreType.DMA,
    ],
)
def _gcn_sc(dst_hbm, src_hbm, h1_hbm, par_hbm,
            out_hbm, hout_hbm,
            g_sp, acc_sp, deg_sp, gt_v, si_v, msg_v, si2_v, msg2_v,
            di_v, ri_v, one_v, z_v, g_v, a_v, h_v, d_v, p_v, gsems, rsems, tsems, sem):
    wid = lax.axis_index("s")
    nsl = pl.ds(wid * RP4, RP4)
    lanes = lax.iota(jnp.int32, 16)

    def _fill(i, _):
        one_v[pl.ds(i * 16, 16)] = jnp.full((16,), 1.0, jnp.float32)
        return _
    lax.fori_loop(0, EC // 16, _fill, None)

    def _zfill(i, _):
        z_v[pl.ds(i * 16, 16)] = jnp.zeros((16,), jnp.float32)
        return _
    lax.fori_loop(0, RP4 // 16, _zfill, None)

    pltpu.sync_copy(par_hbm, p_v)
    pltpu.sync_copy(z_v.at[pl.ds(0, RP)], deg_sp.at[pl.ds(wid * RP, RP)])
    pltpu.sync_copy(z_v, acc_sp.at[nsl])
    plsc.subcore_barrier()

    ddsc = [None, None]
    dstg = [pltpu.async_copy(dst_hbm.at[pl.ds(wid * EPW, EC)], di_v[0],
                             rsems[0])]
    for c in range(NCHUNK):
        b = c % 2
        nb = 1 - b
        if c + 1 < NCHUNK:
            dstg.append(pltpu.async_copy(
                dst_hbm.at[pl.ds(wid * EPW + (c + 1) * EC, EC)], di_v[nb],
                rsems[nb]))
        if ddsc[b] is not None:
            ddsc[b].wait()
        dstg[c].wait()
        ddsc[b] = pltpu.async_copy(one_v, deg_sp.at[di_v[b]], gsems[b],
                                   add=True)
    for d in ddsc:
        if d is not None:
            d.wait()
    plsc.subcore_barrier()

    pltpu.sync_copy(deg_sp.at[pl.ds(wid * RP, RP)], d_v)

    def _dinv(i, _):
        s = pl.ds(i * 16, 16)
        d_v[s] = _rsqrt16(d_v[s] + 1.0)
        return _
    lax.fori_loop(0, RP // 16, _dinv, None)

    pltpu.sync_copy(h1_hbm.at[nsl], g_v)

    def _scale(i, _):
        s = pl.ds(i * 16, 16)
        dv = plsc.load_gather(d_v, [(lanes + i * 16) >> 2])
        g_v[s] = g_v[s] * dv
        return _
    lax.fori_loop(0, RP4 // 16, _scale, None)

    pltpu.sync_copy(g_v, g_sp.at[nsl])
    plsc.subcore_barrier()

    def _edge_pass(si, msg, mult, lidx, lmod):
        iters = EC * mult // 16
        step = 16 // mult

        def esl(c):
            return pl.ds(wid * EPW + c * EC, EC)

        descs = [None, None]
        sdesc = [pltpu.async_copy(src_hbm.at[esl(0)], di_v[0], rsems[0])]
        ddesc = [pltpu.async_copy(dst_hbm.at[esl(0)], ri_v[0], tsems[0])]
        for c in range(NCHUNK):
            b = c % 2
            nb = 1 - b
            if c + 1 < NCHUNK:
                sdesc.append(
                    pltpu.async_copy(src_hbm.at[esl(c + 1)], di_v[nb],
                                     rsems[nb]))
                ddesc.append(
                    pltpu.async_copy(dst_hbm.at[esl(c + 1)], ri_v[nb],
                                     tsems[nb]))
            if descs[b] is not None:
                descs[b].wait()
            sdesc[c].wait()
            ddesc[c].wait()
            sib = si[b]
            msgb = msg[b]
            dib = di_v[b]
            rib = ri_v[b]

            def body(i, _):
                s = pl.ds(i * 16, 16)
                rs = plsc.load_gather(dib, [lidx + i * step])
                msgb[s] = plsc.load_gather(gt_v, [(rs << 2) + lmod])
                rd = plsc.load_gather(rib, [lidx + i * step])
                sib[s] = (rd << 2) + lmod
                return _
            lax.fori_loop(0, iters, body, None)
            descs[b] = pltpu.async_copy(msgb, acc_sp.at[sib], gsems[b],
                                        add=True)
        for d in descs:
            if d is not None:
                d.wait()

    l4 = lanes >> 2
    lm = lanes & 3
    l2 = lanes >> 1
    lm2 = lanes & 1

    for layer in range(3):
        pltpu.sync_copy(g_sp, gt_v)
        if layer < 2:
            _edge_pass(si_v, msg_v, 4, l4, lm)
        else:
            _edge_pass(si2_v, msg2_v, 2, l2, lm2)
        plsc.subcore_barrier()

        pltpu.sync_copy(acc_sp.at[nsl], a_v)
        boff = 48 + 4 * layer

        def _node(i, _):
            s = pl.ds(i * 16, 16)
            flat = lanes + i * 16
            dv = plsc.load_gather(d_v, [flat >> 2])
            bv = plsc.load_gather(p_v, [boff + (flat & 3)])
            h_v[s] = _tanh16(dv * (a_v[s] + g_v[s]) + bv)
            return _
        lax.fori_loop(0, RP4 // 16, _node, None)

        if layer < 2:
            woff = 16 * layer

            def _mm(i, _):
                s = pl.ds(i * 16, 16)
                flat = lanes + i * 16
                nd4 = (flat >> 2) << 2
                cc = flat & 3
                acc = jnp.zeros((16,), jnp.float32)
                for k in range(4):
                    hk = plsc.load_gather(h_v, [nd4 + k])
                    wk = plsc.load_gather(p_v, [woff + 4 * k + cc])
                    acc = acc + hk * wk
                dv = plsc.load_gather(d_v, [flat >> 2])
                g_v[s] = acc * dv
                return _
            lax.fori_loop(0, RP4 // 16, _mm, None)

            pltpu.sync_copy(g_v, g_sp.at[nsl])
            pltpu.sync_copy(z_v, acc_sp.at[nsl])
        else:
            def _cls(i, _):
                s = pl.ds(i * 16, 16)
                flat = lanes + i * 16
                nd4 = (flat >> 2) << 2
                cc = flat & 3
                acc = plsc.load_gather(p_v, [60 + cc])
                for k in range(4):
                    hk = plsc.load_gather(h_v, [nd4 + k])
                    wk = plsc.load_gather(p_v, [32 + 4 * k + cc])
                    acc = acc + hk * wk
                a_v[s] = acc
                return _
            lax.fori_loop(0, RP4 // 16, _cls, None)

            pltpu.sync_copy(a_v, out_hbm.at[nsl])

            def _hout(i, _):
                s = pl.ds(i * 16, 16)
                f2 = lanes + i * 16
                g_v[s] = plsc.load_gather(h_v, [((f2 >> 1) << 2) + (f2 & 1)])
                return _
            lax.fori_loop(0, RP * 2 // 16, _hout, None)

            pltpu.sync_copy(g_v.at[pl.ds(0, RP * 2)],
                            hout_hbm.at[pl.ds(wid * RP * 2, RP * 2)])
        plsc.subcore_barrier()


def kernel(x, edge_index, W1, b1, W2, b2, W3, b3, Wc, bc):
    src = edge_index[0]
    dst = edge_index[1]

    h1 = pl.pallas_call(
        _mm_body,
        out_shape=jax.ShapeDtypeStruct((N, 4), jnp.float32),
    )(x, W1)

    h1f = jnp.pad(h1, ((0, NP - N), (0, 0))).ravel()
    w3p = jnp.pad(W3, ((0, 0), (0, 2)))
    wcp = jnp.pad(Wc, ((0, 2), (0, 0)))
    b3p = jnp.pad(b3, (0, 2))
    par = jnp.concatenate([W2.ravel(), w3p.ravel(), wcp.ravel(),
                           b1, b2, b3p, bc]).astype(jnp.float32)

    out_f, h_f = _gcn_sc(dst, src, h1f, par)
    out = out_f.reshape(NP, 4)[:N]
    h = h_f.reshape(NP, 2)[:N]
    return (out, h)
